# baseline (speedup 1.0000x reference)
import functools

import jax
import jax.numpy as jnp
from jax import lax
from jax.experimental import pallas as pl
from jax.experimental.pallas import tpu as pltpu
from jax.experimental.pallas import tpu_sc as plsc

_B, _T, _DIM, _CDIM, _K = 16, 2048, 512, 64, 1024
_CPAD = 128
_ROWS = _B * _T
_NSLICE = 1
_SROWS = _ROWS // _NSLICE
_RB = 1024
_NBLK = _SROWS // _RB


def _argmax_body(x_ref, wint_ref, bin_ref, embt2_ref, ind_ref):
    z = jnp.dot(x_ref[...], wint_ref[...],
                preferred_element_type=jnp.float32) + bin_ref[...]
    et2 = embt2_ref[...]
    esq = 0.25 * jnp.sum(et2 * et2, axis=0, keepdims=True)
    fe2 = jnp.dot(z, et2, preferred_element_type=jnp.float32)
    flatsq = jnp.sum(z * z, axis=1, keepdims=True)
    a = flatsq - fe2 + esq
    amin = jnp.min(a, axis=1, keepdims=True)
    iota = lax.broadcasted_iota(jnp.int32, (_RB, _K), 1)
    ind_ref[0, 0, :] = jnp.min(jnp.where(a == amin, iota, _K), axis=1)


def _compute_indices(x2d, w_in_t, b_in2d, emb_t):
    ind3 = pl.pallas_call(
        _argmax_body,
        grid=(_NBLK,),
        in_specs=[
            pl.BlockSpec((_RB, _DIM), lambda i: (i, 0)),
            pl.BlockSpec((_DIM, _CDIM), lambda i: (0, 0)),
            pl.BlockSpec((1, _CDIM), lambda i: (0, 0)),
            pl.BlockSpec((_CDIM, _K), lambda i: (0, 0)),
        ],
        out_specs=pl.BlockSpec((1, 1, _RB), lambda i: (i, 0, 0)),
        out_shape=jax.ShapeDtypeStruct((_NBLK, 1, _RB), jnp.int32),
        compiler_params=pltpu.CompilerParams(
            dimension_semantics=("arbitrary",)),
    )(x2d, w_in_t, b_in2d, emb_t)
    return ind3.reshape(_SROWS)


_NC = 2
_NS = 16
_NW = _NC * _NS
_BPW = _SROWS // _NW
_CH = 128
_NCH = _BPW // _CH


def _sc_gather_body(emb_hbm, idx_hbm, q_hbm, idx_v, buf0, buf1, g0, g1):
    wid = lax.axis_index("s") * _NC + lax.axis_index("c")
    base = wid * _BPW
    pltpu.sync_copy(idx_hbm.at[wid], idx_v)
    pltpu.async_copy(emb_hbm.at[idx_v.at[0]], buf0, g0)

    def body(i, _):
        c0 = 2 * i
        pltpu.async_copy(emb_hbm.at[idx_v.at[c0 + 1]], buf1, g1)
        pltpu.make_async_copy(emb_hbm.at[idx_v.at[c0]], buf0, g0).wait()
        pltpu.sync_copy(buf0, q_hbm.at[pl.ds(base + c0 * _CH, _CH)])

        @pl.when(i < _NCH // 2 - 1)
        def _():
            pltpu.async_copy(emb_hbm.at[idx_v.at[c0 + 2]], buf0, g0)

        pltpu.make_async_copy(emb_hbm.at[idx_v.at[c0 + 1]], buf1, g1).wait()
        pltpu.sync_copy(buf1, q_hbm.at[pl.ds(base + (c0 + 1) * _CH, _CH)])
        return 0

    lax.fori_loop(0, _NCH // 2, body, 0)


def _sc_gather(embed_pad, ind3):
    mesh = plsc.VectorSubcoreMesh(core_axis_name="c", subcore_axis_name="s")
    k = functools.partial(
        pl.kernel,
        mesh=mesh,
        out_type=jax.ShapeDtypeStruct((_SROWS, _CPAD), jnp.float32),
        scratch_types=[
            pltpu.VMEM((_NCH, _CH), jnp.int32),
            pltpu.VMEM((_CH, _CPAD), jnp.float32),
            pltpu.VMEM((_CH, _CPAD), jnp.float32),
            pltpu.SemaphoreType.DMA,
            pltpu.SemaphoreType.DMA,
        ],
    )(_sc_gather_body)
    return k(embed_pad, ind3)


def _proj_out_body(q_ref, woutt_ref, bout_ref, out_ref):
    out_ref[...] = jnp.dot(q_ref[...], woutt_ref[...],
                           preferred_element_type=jnp.float32) + bout_ref[...]


def _project_out(q, w_out_t, b_out2d):
    return pl.pallas_call(
        _proj_out_body,
        grid=(_NBLK,),
        in_specs=[
            pl.BlockSpec((_RB, _CPAD), lambda i: (i, 0)),
            pl.BlockSpec((_CPAD, _DIM), lambda i: (0, 0)),
            pl.BlockSpec((1, _DIM), lambda i: (0, 0)),
        ],
        out_specs=pl.BlockSpec((_RB, _DIM), lambda i: (i, 0)),
        out_shape=jax.ShapeDtypeStruct((_SROWS, _DIM), jnp.float32),
        compiler_params=pltpu.CompilerParams(
            dimension_semantics=("arbitrary",)),
    )(q, w_out_t, b_out2d)


def kernel(x, W_in, b_in, W_out, b_out, embed):
    x2d = x.reshape(_ROWS, _DIM)
    w_in_t = W_in.T
    emb_t2 = 2.0 * embed.T
    b_in2d = b_in.reshape(1, _CDIM)
    w_out_t = W_out.T
    b_out2d = b_out.reshape(1, _DIM)

    embed_pad = jnp.pad(embed, ((0, 0), (0, _CPAD - _CDIM)))
    w_out_t_pad = jnp.pad(w_out_t, ((0, _CPAD - _CDIM), (0, 0)))

    ind = _compute_indices(x2d, w_in_t, b_in2d, emb_t2)
    q = _sc_gather(embed_pad, ind.reshape(_NW, _NCH, _CH))
    out = _project_out(q, w_out_t_pad, b_out2d)
    return out.reshape(_B, _T, _DIM)


# speedup vs baseline: 1.1474x; 1.1474x over previous
import functools

import jax
import jax.numpy as jnp
from jax import lax
from jax.experimental import pallas as pl
from jax.experimental.pallas import tpu as pltpu
from jax.experimental.pallas import tpu_sc as plsc

_B, _T, _DIM, _CDIM, _K = 16, 2048, 512, 64, 1024
_CPAD = 128
_ROWS = _B * _T
_NSLICE = 1
_SROWS = _ROWS // _NSLICE
_RB = 1024
_NBLK = _SROWS // _RB


def _argmax_body(x_ref, wint_ref, bin_ref, embt2_ref, ind_ref):
    z = jnp.dot(x_ref[...], wint_ref[...],
                preferred_element_type=jnp.float32) + bin_ref[...]
    et2 = embt2_ref[...]
    esq = 0.25 * jnp.sum(et2 * et2, axis=0, keepdims=True)
    fe2 = jnp.dot(z, et2, preferred_element_type=jnp.float32)
    flatsq = jnp.sum(z * z, axis=1, keepdims=True)
    a = flatsq - fe2 + esq
    ind_ref[0, 0, :] = jnp.argmin(a, axis=1).astype(jnp.int32)


def _compute_indices(x2d, w_in_t, b_in2d, emb_t):
    ind3 = pl.pallas_call(
        _argmax_body,
        grid=(_NBLK,),
        in_specs=[
            pl.BlockSpec((_RB, _DIM), lambda i: (i, 0)),
            pl.BlockSpec((_DIM, _CDIM), lambda i: (0, 0)),
            pl.BlockSpec((1, _CDIM), lambda i: (0, 0)),
            pl.BlockSpec((_CDIM, _K), lambda i: (0, 0)),
        ],
        out_specs=pl.BlockSpec((1, 1, _RB), lambda i: (i, 0, 0)),
        out_shape=jax.ShapeDtypeStruct((_NBLK, 1, _RB), jnp.int32),
        compiler_params=pltpu.CompilerParams(
            dimension_semantics=("arbitrary",)),
    )(x2d, w_in_t, b_in2d, emb_t)
    return ind3.reshape(_SROWS)


_NC = 2
_NS = 16
_NW = _NC * _NS
_BPW = _SROWS // _NW
_CH = 128
_NCH = _BPW // _CH


def _narrow_copy(src, dst):
    def row(r, _):
        for c in range(_CDIM // 16):
            dst[r, pl.ds(c * 16, 16)] = src[r, pl.ds(c * 16, 16)]
        return 0

    lax.fori_loop(0, _CH, row, 0)


def _sc_gather_body(emb_hbm, idx_hbm, q_hbm, idx_v, buf0, buf1, buf64, g0, g1):
    wid = lax.axis_index("s") * _NC + lax.axis_index("c")
    base = wid * _BPW
    pltpu.sync_copy(idx_hbm.at[wid], idx_v)
    pltpu.async_copy(emb_hbm.at[idx_v.at[0]], buf0, g0)

    def body(i, _):
        c0 = 2 * i
        pltpu.async_copy(emb_hbm.at[idx_v.at[c0 + 1]], buf1, g1)
        pltpu.make_async_copy(emb_hbm.at[idx_v.at[c0]], buf0, g0).wait()
        _narrow_copy(buf0, buf64)

        @pl.when(i < _NCH // 2 - 1)
        def _():
            pltpu.async_copy(emb_hbm.at[idx_v.at[c0 + 2]], buf0, g0)

        pltpu.sync_copy(buf64, q_hbm.at[pl.ds(base + c0 * _CH, _CH)])
        pltpu.make_async_copy(emb_hbm.at[idx_v.at[c0 + 1]], buf1, g1).wait()
        _narrow_copy(buf1, buf64)
        pltpu.sync_copy(buf64, q_hbm.at[pl.ds(base + (c0 + 1) * _CH, _CH)])
        return 0

    lax.fori_loop(0, _NCH // 2, body, 0)


def _sc_gather(embed_pad, ind3):
    mesh = plsc.VectorSubcoreMesh(core_axis_name="c", subcore_axis_name="s")
    k = functools.partial(
        pl.kernel,
        mesh=mesh,
        out_type=jax.ShapeDtypeStruct((_SROWS, _CDIM), jnp.float32),
        scratch_types=[
            pltpu.VMEM((_NCH, _CH), jnp.int32),
            pltpu.VMEM((_CH, _CPAD), jnp.float32),
            pltpu.VMEM((_CH, _CPAD), jnp.float32),
            pltpu.VMEM((_CH, _CDIM), jnp.float32),
            pltpu.SemaphoreType.DMA,
            pltpu.SemaphoreType.DMA,
        ],
    )(_sc_gather_body)
    return k(embed_pad, ind3)


def _proj_out_body(q_ref, woutt_ref, bout_ref, out_ref):
    out_ref[...] = jnp.dot(q_ref[...], woutt_ref[...],
                           preferred_element_type=jnp.float32) + bout_ref[...]


def _project_out(q, w_out_t, b_out2d):
    return pl.pallas_call(
        _proj_out_body,
        grid=(_NBLK,),
        in_specs=[
            pl.BlockSpec((_RB, _CDIM), lambda i: (i, 0)),
            pl.BlockSpec((_CDIM, _DIM), lambda i: (0, 0)),
            pl.BlockSpec((1, _DIM), lambda i: (0, 0)),
        ],
        out_specs=pl.BlockSpec((_RB, _DIM), lambda i: (i, 0)),
        out_shape=jax.ShapeDtypeStruct((_SROWS, _DIM), jnp.float32),
        compiler_params=pltpu.CompilerParams(
            dimension_semantics=("arbitrary",)),
    )(q, w_out_t, b_out2d)


def kernel(x, W_in, b_in, W_out, b_out, embed):
    x2d = x.reshape(_ROWS, _DIM)
    w_in_t = W_in.T
    emb_t2 = 2.0 * embed.T
    b_in2d = b_in.reshape(1, _CDIM)
    w_out_t = W_out.T
    b_out2d = b_out.reshape(1, _DIM)

    embed_pad = jnp.pad(embed, ((0, 0), (0, _CPAD - _CDIM)))

    ind = _compute_indices(x2d, w_in_t, b_in2d, emb_t2)
    q = _sc_gather(embed_pad, ind.reshape(_NW, _NCH, _CH))
    out = _project_out(q, w_out_t, b_out2d)
    return out.reshape(_B, _T, _DIM)


# speedup vs baseline: 1.1996x; 1.0455x over previous
import functools

import jax
import jax.numpy as jnp
from jax import lax
from jax.experimental import pallas as pl
from jax.experimental.pallas import tpu as pltpu
from jax.experimental.pallas import tpu_sc as plsc

_B, _T, _DIM, _CDIM, _K = 16, 2048, 512, 64, 1024
_CPAD = 128
_ROWS = _B * _T
_NSLICE = 1
_SROWS = _ROWS // _NSLICE
_RB = 1024
_NBLK = _SROWS // _RB


def _argmax_body(x_ref, wint_ref, bin_ref, emb2_ref, ind_ref):
    z = jnp.dot(x_ref[...], wint_ref[...],
                preferred_element_type=jnp.float32) + bin_ref[...]
    e2 = emb2_ref[...]
    esq_t = 0.25 * jnp.sum(e2 * e2, axis=1, keepdims=True)
    flatsq_row = jnp.sum(z * z, axis=1, keepdims=True).T
    fe2_t = lax.dot_general(e2, z, (((1,), (1,)), ((), ())),
                            preferred_element_type=jnp.float32)
    a_t = flatsq_row - fe2_t + esq_t
    amin = jnp.min(a_t, axis=0)
    iota0 = lax.broadcasted_iota(jnp.int32, (_K, _RB), 0)
    ind_ref[0, 0, :] = jnp.min(jnp.where(a_t == amin[None, :], iota0, _K),
                               axis=0)


def _compute_indices(x2d, w_in_t, b_in2d, emb_t):
    ind3 = pl.pallas_call(
        _argmax_body,
        grid=(_NBLK,),
        in_specs=[
            pl.BlockSpec((_RB, _DIM), lambda i: (i, 0)),
            pl.BlockSpec((_DIM, _CDIM), lambda i: (0, 0)),
            pl.BlockSpec((1, _CDIM), lambda i: (0, 0)),
            pl.BlockSpec((_K, _CDIM), lambda i: (0, 0)),
        ],
        out_specs=pl.BlockSpec((1, 1, _RB), lambda i: (i, 0, 0)),
        out_shape=jax.ShapeDtypeStruct((_NBLK, 1, _RB), jnp.int32),
        compiler_params=pltpu.CompilerParams(
            dimension_semantics=("arbitrary",)),
    )(x2d, w_in_t, b_in2d, emb_t)
    return ind3.reshape(_SROWS)


_NC = 2
_NS = 16
_NW = _NC * _NS
_BPW = _SROWS // _NW
_CH = 128
_NCH = _BPW // _CH


def _narrow_copy(src, dst):
    def row(r, _):
        for c in range(_CDIM // 16):
            dst[r, pl.ds(c * 16, 16)] = src[r, pl.ds(c * 16, 16)]
        return 0

    lax.fori_loop(0, _CH, row, 0)


def _sc_gather_body(emb_hbm, idx_hbm, q_hbm, idx_v, buf0, buf1, buf64, g0, g1):
    wid = lax.axis_index("s") * _NC + lax.axis_index("c")
    base = wid * _BPW
    pltpu.sync_copy(idx_hbm.at[wid], idx_v)
    pltpu.async_copy(emb_hbm.at[idx_v.at[0]], buf0, g0)

    def body(i, _):
        c0 = 2 * i
        pltpu.async_copy(emb_hbm.at[idx_v.at[c0 + 1]], buf1, g1)
        pltpu.make_async_copy(emb_hbm.at[idx_v.at[c0]], buf0, g0).wait()
        _narrow_copy(buf0, buf64)

        @pl.when(i < _NCH // 2 - 1)
        def _():
            pltpu.async_copy(emb_hbm.at[idx_v.at[c0 + 2]], buf0, g0)

        pltpu.sync_copy(buf64, q_hbm.at[pl.ds(base + c0 * _CH, _CH)])
        pltpu.make_async_copy(emb_hbm.at[idx_v.at[c0 + 1]], buf1, g1).wait()
        _narrow_copy(buf1, buf64)
        pltpu.sync_copy(buf64, q_hbm.at[pl.ds(base + (c0 + 1) * _CH, _CH)])
        return 0

    lax.fori_loop(0, _NCH // 2, body, 0)


def _sc_gather(embed_pad, ind3):
    mesh = plsc.VectorSubcoreMesh(core_axis_name="c", subcore_axis_name="s")
    k = functools.partial(
        pl.kernel,
        mesh=mesh,
        out_type=jax.ShapeDtypeStruct((_SROWS, _CDIM), jnp.float32),
        scratch_types=[
            pltpu.VMEM((_NCH, _CH), jnp.int32),
            pltpu.VMEM((_CH, _CPAD), jnp.float32),
            pltpu.VMEM((_CH, _CPAD), jnp.float32),
            pltpu.VMEM((_CH, _CDIM), jnp.float32),
            pltpu.SemaphoreType.DMA,
            pltpu.SemaphoreType.DMA,
        ],
    )(_sc_gather_body)
    return k(embed_pad, ind3)


def _proj_out_body(q_ref, woutt_ref, bout_ref, out_ref):
    out_ref[...] = jnp.dot(q_ref[...], woutt_ref[...],
                           preferred_element_type=jnp.float32) + bout_ref[...]


def _project_out(q, w_out_t, b_out2d):
    return pl.pallas_call(
        _proj_out_body,
        grid=(_NBLK,),
        in_specs=[
            pl.BlockSpec((_RB, _CDIM), lambda i: (i, 0)),
            pl.BlockSpec((_CDIM, _DIM), lambda i: (0, 0)),
            pl.BlockSpec((1, _DIM), lambda i: (0, 0)),
        ],
        out_specs=pl.BlockSpec((_RB, _DIM), lambda i: (i, 0)),
        out_shape=jax.ShapeDtypeStruct((_SROWS, _DIM), jnp.float32),
        compiler_params=pltpu.CompilerParams(
            dimension_semantics=("arbitrary",)),
    )(q, w_out_t, b_out2d)


def kernel(x, W_in, b_in, W_out, b_out, embed):
    x2d = x.reshape(_ROWS, _DIM)
    w_in_t = W_in.T
    emb2 = 2.0 * embed
    b_in2d = b_in.reshape(1, _CDIM)
    w_out_t = W_out.T
    b_out2d = b_out.reshape(1, _DIM)

    embed_pad = jnp.pad(embed, ((0, 0), (0, _CPAD - _CDIM)))

    ind = _compute_indices(x2d, w_in_t, b_in2d, emb2)
    q = _sc_gather(embed_pad, ind.reshape(_NW, _NCH, _CH))
    out = _project_out(q, w_out_t, b_out2d)
    return out.reshape(_B, _T, _DIM)


# speedup vs baseline: 1.3406x; 1.1175x over previous
import functools

import jax
import jax.numpy as jnp
from jax import lax
from jax.experimental import pallas as pl
from jax.experimental.pallas import tpu as pltpu
from jax.experimental.pallas import tpu_sc as plsc

_B, _T, _DIM, _CDIM, _K = 16, 2048, 512, 64, 1024
_CPAD = 128
_ROWS = _B * _T
_NSLICE = 1
_SROWS = _ROWS // _NSLICE
_RB = 2048
_NBLK = _SROWS // _RB


def _argmax_body(x_ref, wint_ref, bin_ref, emb2_ref, ind_ref):
    z = jnp.dot(x_ref[...], wint_ref[...],
                preferred_element_type=jnp.float32) + bin_ref[...]
    e2 = emb2_ref[...]
    esq_t = 0.25 * jnp.sum(e2 * e2, axis=1, keepdims=True)
    flatsq_row = jnp.sum(z * z, axis=1, keepdims=True).T
    fe2_t = lax.dot_general(e2, z, (((1,), (1,)), ((), ())),
                            preferred_element_type=jnp.float32)
    a_t = flatsq_row - fe2_t + esq_t
    amin = jnp.min(a_t, axis=0)
    iota0 = lax.broadcasted_iota(jnp.int32, (_K, _RB), 0)
    ind_ref[0, 0, :] = jnp.min(jnp.where(a_t == amin[None, :], iota0, _K),
                               axis=0)


def _compute_indices(x2d, w_in_t, b_in2d, emb_t):
    ind3 = pl.pallas_call(
        _argmax_body,
        grid=(_NBLK,),
        in_specs=[
            pl.BlockSpec((_RB, _DIM), lambda i: (i, 0)),
            pl.BlockSpec((_DIM, _CDIM), lambda i: (0, 0)),
            pl.BlockSpec((1, _CDIM), lambda i: (0, 0)),
            pl.BlockSpec((_K, _CDIM), lambda i: (0, 0)),
        ],
        out_specs=pl.BlockSpec((1, 1, _RB), lambda i: (i, 0, 0)),
        out_shape=jax.ShapeDtypeStruct((_NBLK, 1, _RB), jnp.int32),
        compiler_params=pltpu.CompilerParams(
            dimension_semantics=("arbitrary",)),
    )(x2d, w_in_t, b_in2d, emb_t)
    return ind3.reshape(_SROWS)


_NC = 2
_NS = 16
_NW = _NC * _NS
_BPW = _SROWS // _NW
_CH = 128
_NCH = _BPW // _CH


def _narrow_copy(src, dst):
    def row(r, _):
        for c in range(_CDIM // 16):
            dst[r, pl.ds(c * 16, 16)] = src[r, pl.ds(c * 16, 16)]
        return 0

    lax.fori_loop(0, _CH, row, 0)


def _sc_gather_body(emb_hbm, idx_hbm, q_hbm, idx_v, buf0, buf1, buf64, g0, g1):
    wid = lax.axis_index("s") * _NC + lax.axis_index("c")
    base = wid * _BPW
    pltpu.sync_copy(idx_hbm.at[wid], idx_v)
    pltpu.async_copy(emb_hbm.at[idx_v.at[0]], buf0, g0)

    def body(i, _):
        c0 = 2 * i
        pltpu.async_copy(emb_hbm.at[idx_v.at[c0 + 1]], buf1, g1)
        pltpu.make_async_copy(emb_hbm.at[idx_v.at[c0]], buf0, g0).wait()
        _narrow_copy(buf0, buf64)

        @pl.when(i < _NCH // 2 - 1)
        def _():
            pltpu.async_copy(emb_hbm.at[idx_v.at[c0 + 2]], buf0, g0)

        pltpu.sync_copy(buf64, q_hbm.at[pl.ds(base + c0 * _CH, _CH)])
        pltpu.make_async_copy(emb_hbm.at[idx_v.at[c0 + 1]], buf1, g1).wait()
        _narrow_copy(buf1, buf64)
        pltpu.sync_copy(buf64, q_hbm.at[pl.ds(base + (c0 + 1) * _CH, _CH)])
        return 0

    lax.fori_loop(0, _NCH // 2, body, 0)


def _sc_gather(embed_pad, ind3):
    mesh = plsc.VectorSubcoreMesh(core_axis_name="c", subcore_axis_name="s")
    k = functools.partial(
        pl.kernel,
        mesh=mesh,
        out_type=jax.ShapeDtypeStruct((_SROWS, _CDIM), jnp.float32),
        scratch_types=[
            pltpu.VMEM((_NCH, _CH), jnp.int32),
            pltpu.VMEM((_CH, _CPAD), jnp.float32),
            pltpu.VMEM((_CH, _CPAD), jnp.float32),
            pltpu.VMEM((_CH, _CDIM), jnp.float32),
            pltpu.SemaphoreType.DMA,
            pltpu.SemaphoreType.DMA,
        ],
    )(_sc_gather_body)
    return k(embed_pad, ind3)


def _proj_out_body(q_ref, woutt_ref, bout_ref, out_ref):
    out_ref[...] = jnp.dot(q_ref[...], woutt_ref[...],
                           preferred_element_type=jnp.float32) + bout_ref[...]


def _project_out(q, w_out_t, b_out2d):
    return pl.pallas_call(
        _proj_out_body,
        grid=(_NBLK,),
        in_specs=[
            pl.BlockSpec((_RB, _CDIM), lambda i: (i, 0)),
            pl.BlockSpec((_CDIM, _DIM), lambda i: (0, 0)),
            pl.BlockSpec((1, _DIM), lambda i: (0, 0)),
        ],
        out_specs=pl.BlockSpec((_RB, _DIM), lambda i: (i, 0)),
        out_shape=jax.ShapeDtypeStruct((_SROWS, _DIM), jnp.float32),
        compiler_params=pltpu.CompilerParams(
            dimension_semantics=("arbitrary",)),
    )(q, w_out_t, b_out2d)


def kernel(x, W_in, b_in, W_out, b_out, embed):
    x2d = x.reshape(_ROWS, _DIM)
    w_in_t = W_in.T
    emb2 = 2.0 * embed
    b_in2d = b_in.reshape(1, _CDIM)
    w_out_t = W_out.T
    b_out2d = b_out.reshape(1, _DIM)

    embed_pad = jnp.pad(embed, ((0, 0), (0, _CPAD - _CDIM)))

    ind = _compute_indices(x2d, w_in_t, b_in2d, emb2)
    q = _sc_gather(embed_pad, ind.reshape(_NW, _NCH, _CH))
    out = _project_out(q, w_out_t, b_out2d)
    return out.reshape(_B, _T, _DIM)


# speedup vs baseline: 1.3685x; 1.0208x over previous
import functools

import jax
import jax.numpy as jnp
from jax import lax
from jax.experimental import pallas as pl
from jax.experimental.pallas import tpu as pltpu
from jax.experimental.pallas import tpu_sc as plsc

_B, _T, _DIM, _CDIM, _K = 16, 2048, 512, 64, 1024
_CPAD = 128
_ROWS = _B * _T
_NSLICE = 1
_SROWS = _ROWS // _NSLICE
_RB = 4096
_NBLK = _SROWS // _RB


def _argmax_body(x_ref, wint_ref, bin_ref, emb2_ref, ind_ref):
    z = jnp.dot(x_ref[...], wint_ref[...],
                preferred_element_type=jnp.float32) + bin_ref[...]
    e2 = emb2_ref[...]
    esq_t = 0.25 * jnp.sum(e2 * e2, axis=1, keepdims=True)
    flatsq_row = jnp.sum(z * z, axis=1, keepdims=True).T
    fe2_t = lax.dot_general(e2, z, (((1,), (1,)), ((), ())),
                            preferred_element_type=jnp.float32)
    a_t = flatsq_row - fe2_t + esq_t
    amin = jnp.min(a_t, axis=0)
    iota0 = lax.broadcasted_iota(jnp.int32, (_K, _RB), 0)
    ind_ref[0, 0, :] = jnp.min(jnp.where(a_t == amin[None, :], iota0, _K),
                               axis=0)


def _compute_indices(x2d, w_in_t, b_in2d, emb_t):
    ind3 = pl.pallas_call(
        _argmax_body,
        grid=(_NBLK,),
        in_specs=[
            pl.BlockSpec((_RB, _DIM), lambda i: (i, 0)),
            pl.BlockSpec((_DIM, _CDIM), lambda i: (0, 0)),
            pl.BlockSpec((1, _CDIM), lambda i: (0, 0)),
            pl.BlockSpec((_K, _CDIM), lambda i: (0, 0)),
        ],
        out_specs=pl.BlockSpec((1, 1, _RB), lambda i: (i, 0, 0)),
        out_shape=jax.ShapeDtypeStruct((_NBLK, 1, _RB), jnp.int32),
        compiler_params=pltpu.CompilerParams(
            dimension_semantics=("arbitrary",)),
    )(x2d, w_in_t, b_in2d, emb_t)
    return ind3.reshape(_SROWS)


_NC = 2
_NS = 16
_NW = _NC * _NS
_BPW = _SROWS // _NW
_CH = 128
_NCH = _BPW // _CH


def _narrow_copy(src, dst):
    def row(r, _):
        for c in range(_CDIM // 16):
            dst[r, pl.ds(c * 16, 16)] = src[r, pl.ds(c * 16, 16)]
        return 0

    lax.fori_loop(0, _CH, row, 0)


def _sc_gather_body(emb_hbm, idx_hbm, q_hbm, idx_v, buf0, buf1, buf64, g0, g1):
    wid = lax.axis_index("s") * _NC + lax.axis_index("c")
    base = wid * _BPW
    pltpu.sync_copy(idx_hbm.at[wid], idx_v)
    pltpu.async_copy(emb_hbm.at[idx_v.at[0]], buf0, g0)

    def body(i, _):
        c0 = 2 * i
        pltpu.async_copy(emb_hbm.at[idx_v.at[c0 + 1]], buf1, g1)
        pltpu.make_async_copy(emb_hbm.at[idx_v.at[c0]], buf0, g0).wait()
        _narrow_copy(buf0, buf64)

        @pl.when(i < _NCH // 2 - 1)
        def _():
            pltpu.async_copy(emb_hbm.at[idx_v.at[c0 + 2]], buf0, g0)

        pltpu.sync_copy(buf64, q_hbm.at[pl.ds(base + c0 * _CH, _CH)])
        pltpu.make_async_copy(emb_hbm.at[idx_v.at[c0 + 1]], buf1, g1).wait()
        _narrow_copy(buf1, buf64)
        pltpu.sync_copy(buf64, q_hbm.at[pl.ds(base + (c0 + 1) * _CH, _CH)])
        return 0

    lax.fori_loop(0, _NCH // 2, body, 0)


def _sc_gather(embed_pad, ind3):
    mesh = plsc.VectorSubcoreMesh(core_axis_name="c", subcore_axis_name="s")
    k = functools.partial(
        pl.kernel,
        mesh=mesh,
        out_type=jax.ShapeDtypeStruct((_SROWS, _CDIM), jnp.float32),
        scratch_types=[
            pltpu.VMEM((_NCH, _CH), jnp.int32),
            pltpu.VMEM((_CH, _CPAD), jnp.float32),
            pltpu.VMEM((_CH, _CPAD), jnp.float32),
            pltpu.VMEM((_CH, _CDIM), jnp.float32),
            pltpu.SemaphoreType.DMA,
            pltpu.SemaphoreType.DMA,
        ],
    )(_sc_gather_body)
    return k(embed_pad, ind3)


def _proj_out_body(q_ref, woutt_ref, bout_ref, out_ref):
    out_ref[...] = jnp.dot(q_ref[...], woutt_ref[...],
                           preferred_element_type=jnp.float32) + bout_ref[...]


def _project_out(q, w_out_t, b_out2d):
    return pl.pallas_call(
        _proj_out_body,
        grid=(_NBLK,),
        in_specs=[
            pl.BlockSpec((_RB, _CDIM), lambda i: (i, 0)),
            pl.BlockSpec((_CDIM, _DIM), lambda i: (0, 0)),
            pl.BlockSpec((1, _DIM), lambda i: (0, 0)),
        ],
        out_specs=pl.BlockSpec((_RB, _DIM), lambda i: (i, 0)),
        out_shape=jax.ShapeDtypeStruct((_SROWS, _DIM), jnp.float32),
        compiler_params=pltpu.CompilerParams(
            dimension_semantics=("arbitrary",)),
    )(q, w_out_t, b_out2d)


def kernel(x, W_in, b_in, W_out, b_out, embed):
    x2d = x.reshape(_ROWS, _DIM)
    w_in_t = W_in.T
    emb2 = 2.0 * embed
    b_in2d = b_in.reshape(1, _CDIM)
    w_out_t = W_out.T
    b_out2d = b_out.reshape(1, _DIM)

    embed_pad = jnp.pad(embed, ((0, 0), (0, _CPAD - _CDIM)))

    ind = _compute_indices(x2d, w_in_t, b_in2d, emb2)
    q = _sc_gather(embed_pad, ind.reshape(_NW, _NCH, _CH))
    out = _project_out(q, w_out_t, b_out2d)
    return out.reshape(_B, _T, _DIM)


# speedup vs baseline: 1.5603x; 1.1402x over previous
import functools

import jax
import jax.numpy as jnp
from jax import lax
from jax.experimental import pallas as pl
from jax.experimental.pallas import tpu as pltpu
from jax.experimental.pallas import tpu_sc as plsc

_B, _T, _DIM, _CDIM, _K = 16, 2048, 512, 64, 1024
_CPAD = 128
_ROWS = _B * _T
_NSLICE = 1
_SROWS = _ROWS // _NSLICE
_RB = 4096
_NBLK = _SROWS // _RB


def _argmax_body(x_ref, wint_ref, bin_ref, emb2_ref, ind_ref):
    z = jnp.dot(x_ref[...], wint_ref[...],
                preferred_element_type=jnp.float32) + bin_ref[...]
    e2 = emb2_ref[...]
    esq_t = 0.25 * jnp.sum(e2 * e2, axis=1, keepdims=True)
    flatsq_row = jnp.sum(z * z, axis=1, keepdims=True).T
    fe2_t = lax.dot_general(e2, z, (((1,), (1,)), ((), ())),
                            preferred_element_type=jnp.float32)
    a_t = flatsq_row - fe2_t + esq_t
    amin = jnp.min(a_t, axis=0)
    iota0 = lax.broadcasted_iota(jnp.int32, (_K, _RB), 0)
    ind_ref[0, 0, :] = jnp.min(jnp.where(a_t == amin[None, :], iota0, _K),
                               axis=0)


def _compute_indices(x2d, w_in_t, b_in2d, emb_t):
    ind3 = pl.pallas_call(
        _argmax_body,
        grid=(_NBLK,),
        in_specs=[
            pl.BlockSpec((_RB, _DIM), lambda i: (i, 0)),
            pl.BlockSpec((_DIM, _CDIM), lambda i: (0, 0)),
            pl.BlockSpec((1, _CDIM), lambda i: (0, 0)),
            pl.BlockSpec((_K, _CDIM), lambda i: (0, 0)),
        ],
        out_specs=pl.BlockSpec((1, 1, _RB), lambda i: (i, 0, 0)),
        out_shape=jax.ShapeDtypeStruct((_NBLK, 1, _RB), jnp.int32),
        compiler_params=pltpu.CompilerParams(
            dimension_semantics=("arbitrary",)),
    )(x2d, w_in_t, b_in2d, emb_t)
    return ind3.reshape(_SROWS)


_NC = 2
_NS = 16
_NW = _NC * _NS
_BPW = _SROWS // _NW
_CH = 128
_NCH = _BPW // _CH


def _narrow_copy(src, dst):
    def row(r, _):
        for c in range(_CDIM // 16):
            dst[r, pl.ds(c * 16, 16)] = src[r, pl.ds(c * 16, 16)]
        return 0

    lax.fori_loop(0, _CH, row, 0)


def _sc_gather_body(emb_hbm, idx_hbm, q_hbm, idx_v, buf0, buf1, g0, g1):
    wid = lax.axis_index("s") * _NC + lax.axis_index("c")
    base = wid * _BPW
    pltpu.sync_copy(idx_hbm.at[wid], idx_v)
    pltpu.async_copy(emb_hbm.at[idx_v.at[0]], buf0, g0)

    def body(i, _):
        c0 = 2 * i
        pltpu.async_copy(emb_hbm.at[idx_v.at[c0 + 1]], buf1, g1)
        pltpu.make_async_copy(emb_hbm.at[idx_v.at[c0]], buf0, g0).wait()
        pltpu.sync_copy(buf0, q_hbm.at[pl.ds(base + c0 * _CH, _CH)])

        @pl.when(i < _NCH // 2 - 1)
        def _():
            pltpu.async_copy(emb_hbm.at[idx_v.at[c0 + 2]], buf0, g0)

        pltpu.make_async_copy(emb_hbm.at[idx_v.at[c0 + 1]], buf1, g1).wait()
        pltpu.sync_copy(buf1, q_hbm.at[pl.ds(base + (c0 + 1) * _CH, _CH)])
        return 0

    lax.fori_loop(0, _NCH // 2, body, 0)


def _sc_gather(embed64, ind3):
    mesh = plsc.VectorSubcoreMesh(core_axis_name="c", subcore_axis_name="s")
    k = functools.partial(
        pl.kernel,
        mesh=mesh,
        out_type=jax.ShapeDtypeStruct((_SROWS, _CDIM), jnp.float32),
        scratch_types=[
            pltpu.VMEM((_NCH, _CH), jnp.int32),
            pltpu.VMEM((_CH, _CDIM), jnp.float32),
            pltpu.VMEM((_CH, _CDIM), jnp.float32),
            pltpu.SemaphoreType.DMA,
            pltpu.SemaphoreType.DMA,
        ],
        compiler_params=pltpu.CompilerParams(use_tc_tiling_on_sc=False),
    )(_sc_gather_body)
    return k(embed64, ind3)


def _proj_out_body(q_ref, woutt_ref, bout_ref, out_ref):
    out_ref[...] = jnp.dot(q_ref[...], woutt_ref[...],
                           preferred_element_type=jnp.float32) + bout_ref[...]


def _project_out(q, w_out_t, b_out2d):
    return pl.pallas_call(
        _proj_out_body,
        grid=(_NBLK,),
        in_specs=[
            pl.BlockSpec((_RB, _CDIM), lambda i: (i, 0)),
            pl.BlockSpec((_CDIM, _DIM), lambda i: (0, 0)),
            pl.BlockSpec((1, _DIM), lambda i: (0, 0)),
        ],
        out_specs=pl.BlockSpec((_RB, _DIM), lambda i: (i, 0)),
        out_shape=jax.ShapeDtypeStruct((_SROWS, _DIM), jnp.float32),
        compiler_params=pltpu.CompilerParams(
            dimension_semantics=("arbitrary",)),
    )(q, w_out_t, b_out2d)


def kernel(x, W_in, b_in, W_out, b_out, embed):
    x2d = x.reshape(_ROWS, _DIM)
    w_in_t = W_in.T
    emb2 = 2.0 * embed
    b_in2d = b_in.reshape(1, _CDIM)
    w_out_t = W_out.T
    b_out2d = b_out.reshape(1, _DIM)

    ind = _compute_indices(x2d, w_in_t, b_in2d, emb2)
    q = _sc_gather(embed, ind.reshape(_NW, _NCH, _CH))
    out = _project_out(q, w_out_t, b_out2d)
    return out.reshape(_B, _T, _DIM)


# speedup vs baseline: 1.5617x; 1.0009x over previous
import functools

import jax
import jax.numpy as jnp
from jax import lax
from jax.experimental import pallas as pl
from jax.experimental.pallas import tpu as pltpu
from jax.experimental.pallas import tpu_sc as plsc

_B, _T, _DIM, _CDIM, _K = 16, 2048, 512, 64, 1024
_ROWS = _B * _T
_RB = 4096
_NBLK = _ROWS // _RB


def _argmax_body(x_ref, wint_ref, bin_ref, emb2_ref, ind_ref):
    z = jnp.dot(x_ref[...], wint_ref[...],
                preferred_element_type=jnp.float32) + bin_ref[...]
    e2 = emb2_ref[...]
    esq_t = 0.25 * jnp.sum(e2 * e2, axis=1, keepdims=True)
    flatsq_row = jnp.sum(z * z, axis=1, keepdims=True).T
    fe2_t = lax.dot_general(e2, z, (((1,), (1,)), ((), ())),
                            preferred_element_type=jnp.float32)
    a_t = flatsq_row - fe2_t + esq_t
    amin = jnp.min(a_t, axis=0)
    iota0 = lax.broadcasted_iota(jnp.int32, (_K, _RB), 0)
    ind_ref[0, 0, :] = jnp.min(jnp.where(a_t == amin[None, :], iota0, _K),
                               axis=0)


def _compute_indices(x2d, w_in_t, b_in2d, emb_t):
    ind3 = pl.pallas_call(
        _argmax_body,
        grid=(_NBLK,),
        in_specs=[
            pl.BlockSpec((_RB, _DIM), lambda i: (i, 0)),
            pl.BlockSpec((_DIM, _CDIM), lambda i: (0, 0)),
            pl.BlockSpec((1, _CDIM), lambda i: (0, 0)),
            pl.BlockSpec((_K, _CDIM), lambda i: (0, 0)),
        ],
        out_specs=pl.BlockSpec((1, 1, _RB), lambda i: (i, 0, 0)),
        out_shape=jax.ShapeDtypeStruct((_NBLK, 1, _RB), jnp.int32),
        compiler_params=pltpu.CompilerParams(
            dimension_semantics=("arbitrary",)),
    )(x2d, w_in_t, b_in2d, emb_t)
    return ind3.reshape(_ROWS)


_NC = 2
_NS = 16
_NW = _NC * _NS
_BPW = _ROWS // _NW
_CH = 128
_NCH = _BPW // _CH


def _sc_gather_body(emb_hbm, idx_hbm, q_hbm, idx_v, buf0, buf1, g0, g1):
    wid = lax.axis_index("s") * _NC + lax.axis_index("c")
    base = wid * _BPW
    pltpu.sync_copy(idx_hbm.at[wid], idx_v)
    pltpu.async_copy(emb_hbm.at[idx_v.at[0]], buf0, g0)

    def body(i, _):
        c0 = 2 * i
        pltpu.async_copy(emb_hbm.at[idx_v.at[c0 + 1]], buf1, g1)
        pltpu.make_async_copy(emb_hbm.at[idx_v.at[c0]], buf0, g0).wait()
        pltpu.sync_copy(buf0, q_hbm.at[pl.ds(base + c0 * _CH, _CH)])

        @pl.when(i < _NCH // 2 - 1)
        def _():
            pltpu.async_copy(emb_hbm.at[idx_v.at[c0 + 2]], buf0, g0)

        pltpu.make_async_copy(emb_hbm.at[idx_v.at[c0 + 1]], buf1, g1).wait()
        pltpu.sync_copy(buf1, q_hbm.at[pl.ds(base + (c0 + 1) * _CH, _CH)])
        return 0

    lax.fori_loop(0, _NCH // 2, body, 0)


def _sc_gather(embed64, ind3):
    mesh = plsc.VectorSubcoreMesh(core_axis_name="c", subcore_axis_name="s")
    k = functools.partial(
        pl.kernel,
        mesh=mesh,
        out_type=jax.ShapeDtypeStruct((_ROWS, _CDIM), jnp.float32),
        scratch_types=[
            pltpu.VMEM((_NCH, _CH), jnp.int32),
            pltpu.VMEM((_CH, _CDIM), jnp.float32),
            pltpu.VMEM((_CH, _CDIM), jnp.float32),
            pltpu.SemaphoreType.DMA,
            pltpu.SemaphoreType.DMA,
        ],
        compiler_params=pltpu.CompilerParams(use_tc_tiling_on_sc=False),
    )(_sc_gather_body)
    return k(embed64, ind3)


def _proj_out_body(q_ref, woutt_ref, bout_ref, out_ref):
    out_ref[...] = jnp.dot(q_ref[...], woutt_ref[...],
                           preferred_element_type=jnp.float32) + bout_ref[...]


def _project_out(q, w_out_t, b_out2d):
    return pl.pallas_call(
        _proj_out_body,
        grid=(_NBLK,),
        in_specs=[
            pl.BlockSpec((_RB, _CDIM), lambda i: (i, 0)),
            pl.BlockSpec((_CDIM, _DIM), lambda i: (0, 0)),
            pl.BlockSpec((1, _DIM), lambda i: (0, 0)),
        ],
        out_specs=pl.BlockSpec((_RB, _DIM), lambda i: (i, 0)),
        out_shape=jax.ShapeDtypeStruct((_ROWS, _DIM), jnp.float32),
        compiler_params=pltpu.CompilerParams(
            dimension_semantics=("arbitrary",)),
    )(q, w_out_t, b_out2d)


def kernel(x, W_in, b_in, W_out, b_out, embed):
    x2d = x.reshape(_ROWS, _DIM)
    w_in_t = W_in.T
    emb2 = 2.0 * embed
    b_in2d = b_in.reshape(1, _CDIM)
    w_out_t = W_out.T
    b_out2d = b_out.reshape(1, _DIM)

    ind = _compute_indices(x2d, w_in_t, b_in2d, emb2)
    q = _sc_gather(embed, ind.reshape(_NW, _NCH, _CH))
    out = _project_out(q, w_out_t, b_out2d)
    return out.reshape(_B, _T, _DIM)
